# baseline (speedup 1.0000x reference)
import functools

import jax
import jax.numpy as jnp
from jax import lax
from jax.experimental import pallas as pl
from jax.experimental.pallas import tpu as pltpu
from jax.experimental.pallas import tpu_sc as plsc

L = 16
NC = 2
NS = 16
NW = NC * NS
BIG = 0x3FFFFFFF
CHUNK = 10000


def _tile_body(preds_hbm, lits_hbm, clauses_hbm, out_hbm,
               preds_local, packed_local, packed_v, preds_sh, packed_sh,
               lits_b0, lits_b1, cls_b0, cls_b1, outbuf_v,
               sem_l0, sem_l1, sem_c0, sem_c1,
               *, n_vars, per_tile, wpt):
    sid = lax.axis_index("s")
    wid = sid * NC + lax.axis_index("c")
    half = per_tile // 2
    base_a = wid * per_tile
    base_b = base_a + half
    nchunk = half // CHUNK
    lits_b = (lits_b0, lits_b1)
    cls_b = (cls_b0, cls_b1)
    sem_l = (sem_l0, sem_l1)
    sem_c = (sem_c0, sem_c1)

    def start_chunk(slot, c):
        for src, bufs, sems in ((lits_hbm, lits_b, sem_l),
                                (clauses_hbm, cls_b, sem_c)):
            pltpu.make_async_copy(src.at[pl.ds(base_a + c * CHUNK, CHUNK)],
                                  bufs[slot].at[pl.ds(0, CHUNK)],
                                  sems[slot]).start()
            pltpu.make_async_copy(src.at[pl.ds(base_b + c * CHUNK, CHUNK)],
                                  bufs[slot].at[pl.ds(CHUNK, CHUNK)],
                                  sems[slot]).start()

    def wait_chunk(slot):
        for src, bufs, sems in ((lits_hbm, lits_b, sem_l),
                                (clauses_hbm, cls_b, sem_c)):
            pltpu.make_async_copy(src.at[pl.ds(0, CHUNK)],
                                  bufs[slot].at[pl.ds(0, CHUNK)],
                                  sems[slot]).wait()
            pltpu.make_async_copy(src.at[pl.ds(0, CHUNK)],
                                  bufs[slot].at[pl.ds(CHUNK, CHUNK)],
                                  sems[slot]).wait()

    start_chunk(0, 0)
    start_chunk(1, 1)

    iota = lax.iota(jnp.int32, L)
    vpt = 32 * wpt
    groups = wpt // L

    @pl.when(sid == 0)
    def _():
        pltpu.sync_copy(preds_hbm, preds_sh)

    plsc.subcore_barrier()
    tail = n_vars - vpt * (NS - 1)
    assert 0 < tail <= vpt and tail % 8 == 0

    @pl.when(sid < NS - 1)
    def _():
        pltpu.sync_copy(preds_sh.at[pl.ds(vpt * sid, vpt)], preds_local)

    @pl.when(sid == NS - 1)
    def _():
        pltpu.sync_copy(preds_sh.at[pl.ds(vpt * (NS - 1), tail)],
                        preds_local.at[pl.ds(0, tail)])

    powlo = jnp.left_shift(jnp.full((L,), 1, jnp.int32), iota)
    powhi = jnp.left_shift(jnp.full((L,), 1, jnp.int32), iota + L)

    def grp(g, carry):
        acc = jnp.zeros((L,), jnp.int32)
        for j in range(L):
            off = 2 * L * L * g + 2 * L * j
            b0 = preds_local[pl.ds(off, L)] >= 0.5
            b1 = preds_local[pl.ds(off + L, L)] >= 0.5
            w = (jnp.sum(jnp.where(b0, powlo, 0))
                 + jnp.sum(jnp.where(b1, powhi, 0)))
            acc = jnp.where(iota == j, w, acc)
        packed_local[pl.ds(L * g, L)] = acc
        return carry

    lax.fori_loop(0, groups, grp, 0)
    pltpu.sync_copy(packed_local, packed_sh.at[pl.ds(wpt * sid, wpt)])
    plsc.subcore_barrier()
    pltpu.sync_copy(packed_sh, packed_v)

    shift_idx = jnp.maximum(iota - 1, 0)
    last_idx = jnp.full((L,), L - 1, jnp.int32)

    def stream_step(slot, off, i, carry_vec, cnt_vec, first_vec):
        lit = lits_b[slot][pl.ds(off + i * L, L)]
        cls = cls_b[slot][pl.ds(off + i * L, L)]
        is_pos = lit < n_vars
        var = jnp.where(is_pos, lit, lit - n_vars)
        wd = plsc.load_gather(packed_v, [jnp.right_shift(var, 5)])
        bit = jnp.right_shift(wd, var & 31) & 1
        sat = (bit != 0) == is_pos
        m = jnp.where(sat, cls, -1)
        incl = plsc.cummax(m)
        shifted = jnp.take_along_axis(incl, shift_idx, axis=0,
                                      mode="promise_in_bounds")
        shifted = jnp.where(iota == 0, -1, shifted)
        excl = jnp.maximum(shifted, carry_vec)
        newc = sat & (cls > excl)
        cnt_vec = cnt_vec + newc.astype(jnp.int32)
        first_vec = jnp.minimum(first_vec, jnp.where(sat, cls, BIG))
        vmax = jnp.take_along_axis(incl, last_idx, axis=0,
                                   mode="promise_in_bounds")
        carry_vec = jnp.maximum(carry_vec, vmax)
        return carry_vec, cnt_vec, first_vec

    def compute(slot, state):
        def vec_body(i, st):
            ca, cb, cnt_vec, fa, fb = st
            ca, cnt_vec, fa = stream_step(slot, 0, i, ca, cnt_vec, fa)
            cb, cnt_vec, fb = stream_step(slot, CHUNK, i, cb, cnt_vec, fb)
            return ca, cb, cnt_vec, fa, fb

        return lax.fori_loop(0, CHUNK // L, vec_body, state, unroll=4)

    def one(c, slot, state):
        wait_chunk(slot)
        state = compute(slot, state)

        @pl.when(c + 2 < nchunk)
        def _():
            start_chunk(slot, c + 2)

        return state

    def pair_body(i, state):
        state = one(2 * i, 0, state)
        state = one(2 * i + 1, 1, state)
        return state

    init = (jnp.full((L,), -1, jnp.int32),
            jnp.full((L,), -1, jnp.int32),
            jnp.zeros((L,), jnp.int32),
            jnp.full((L,), BIG, jnp.int32),
            jnp.full((L,), BIG, jnp.int32))
    state = lax.fori_loop(0, nchunk // 2, pair_body, init)
    if nchunk % 2:
        state = one(nchunk - 1, 0, state)
    ca, cb, cnt_vec, fa, fb = state

    cnt = jnp.sum(cnt_vec)
    first_a = jnp.min(fa)
    last_a = jnp.max(ca)
    first_b = jnp.min(fb)
    last_b = jnp.max(cb)
    out = jnp.where(iota == 0, cnt,
                    jnp.where(iota == 1, first_a,
                              jnp.where(iota == 2, last_a,
                                        jnp.where(iota == 3, first_b,
                                                  jnp.where(iota == 4,
                                                            last_b, 0)))))
    outbuf_v[...] = out
    pltpu.sync_copy(outbuf_v, out_hbm.at[wid])


def _combine_body(n_vars, partials_ref, ncl_ref, o_ref):
    def body(t, st):
        total, m = st
        total = total + partials_ref[t, 0]
        fa = partials_ref[t, 1]
        la = partials_ref[t, 2]
        fb = partials_ref[t, 3]
        lb = partials_ref[t, 4]
        total = total - jnp.where(fa == m, jnp.int32(1), jnp.int32(0))
        m = jnp.maximum(m, la)
        total = total - jnp.where(fb == m, jnp.int32(1), jnp.int32(0))
        m = jnp.maximum(m, lb)
        return total, m

    total, _ = lax.fori_loop(0, NW, body, (jnp.int32(0), jnp.int32(-1)))
    o_ref[0, 0] = (ncl_ref[0, 0] - total.astype(jnp.float32)) / jnp.float32(n_vars)


def kernel(preds, lits, clauses, n_vars, n_clauses):
    del n_vars
    nv = preds.shape[0]
    nnz = lits.shape[0]
    per_tile = nnz // NW
    assert nnz % NW == 0 and per_tile % 2 == 0
    assert (per_tile // 2) % CHUNK == 0 and CHUNK % L == 0
    nwords = -(-nv // 32)
    wpt = -(-nwords // NS)
    wpt = -(-wpt // L) * L

    mesh = plsc.VectorSubcoreMesh(core_axis_name="c", subcore_axis_name="s")
    sc = functools.partial(
        pl.kernel,
        mesh=mesh,
        compiler_params=pltpu.CompilerParams(needs_layout_passes=False),
        out_type=jax.ShapeDtypeStruct((NW, L), jnp.int32),
        scratch_types=[
            pltpu.VMEM((32 * wpt,), jnp.float32),
            pltpu.VMEM((wpt,), jnp.int32),
            pltpu.VMEM((NS * wpt,), jnp.int32),
            pltpu.VMEM_SHARED((nv,), jnp.float32),
            pltpu.VMEM_SHARED((NS * wpt,), jnp.int32),
            pltpu.VMEM((2 * CHUNK,), jnp.int32),
            pltpu.VMEM((2 * CHUNK,), jnp.int32),
            pltpu.VMEM((2 * CHUNK,), jnp.int32),
            pltpu.VMEM((2 * CHUNK,), jnp.int32),
            pltpu.VMEM((L,), jnp.int32),
            pltpu.SemaphoreType.DMA,
            pltpu.SemaphoreType.DMA,
            pltpu.SemaphoreType.DMA,
            pltpu.SemaphoreType.DMA,
        ],
    )(functools.partial(_tile_body, n_vars=nv, per_tile=per_tile, wpt=wpt))
    partials = sc(preds, lits, clauses)

    ncl = jnp.asarray(n_clauses, jnp.float32).reshape(1, 1)
    out = pl.pallas_call(
        functools.partial(_combine_body, nv),
        in_specs=[pl.BlockSpec(memory_space=pltpu.SMEM),
                  pl.BlockSpec(memory_space=pltpu.SMEM)],
        out_specs=pl.BlockSpec(memory_space=pltpu.SMEM),
        out_shape=jax.ShapeDtypeStruct((1, 1), jnp.float32),
    )(partials, ncl)
    return out[0, 0]


# speedup vs baseline: 1.0158x; 1.0158x over previous
import functools

import jax
import jax.numpy as jnp
from jax import lax
from jax.experimental import pallas as pl
from jax.experimental.pallas import tpu as pltpu
from jax.experimental.pallas import tpu_sc as plsc

L = 16
NC = 2
NS = 16
NW = NC * NS
BIG = 0x3FFFFFFF
CHUNK = 2000


def _tile_body(preds_hbm, lits_hbm, clauses_hbm, out_hbm,
               preds_local, packed_local, packed_v, preds_sh, packed_sh,
               lits_b0, lits_b1, cls_b0, cls_b1, outbuf_v,
               sem_l0, sem_l1, sem_c0, sem_c1,
               *, n_vars, per_tile, wpt):
    sid = lax.axis_index("s")
    wid = sid * NC + lax.axis_index("c")
    half = per_tile // 2
    base_a = wid * per_tile
    base_b = base_a + half
    nchunk = half // CHUNK
    lits_b = (lits_b0, lits_b1)
    cls_b = (cls_b0, cls_b1)
    sem_l = (sem_l0, sem_l1)
    sem_c = (sem_c0, sem_c1)

    def start_chunk(slot, c):
        for src, bufs, sems in ((lits_hbm, lits_b, sem_l),
                                (clauses_hbm, cls_b, sem_c)):
            pltpu.make_async_copy(src.at[pl.ds(base_a + c * CHUNK, CHUNK)],
                                  bufs[slot].at[pl.ds(0, CHUNK)],
                                  sems[slot]).start()
            pltpu.make_async_copy(src.at[pl.ds(base_b + c * CHUNK, CHUNK)],
                                  bufs[slot].at[pl.ds(CHUNK, CHUNK)],
                                  sems[slot]).start()

    def wait_chunk(slot):
        for src, bufs, sems in ((lits_hbm, lits_b, sem_l),
                                (clauses_hbm, cls_b, sem_c)):
            pltpu.make_async_copy(src.at[pl.ds(0, CHUNK)],
                                  bufs[slot].at[pl.ds(0, CHUNK)],
                                  sems[slot]).wait()
            pltpu.make_async_copy(src.at[pl.ds(0, CHUNK)],
                                  bufs[slot].at[pl.ds(CHUNK, CHUNK)],
                                  sems[slot]).wait()

    start_chunk(0, 0)
    start_chunk(1, 1)

    iota = lax.iota(jnp.int32, L)
    vpt = 32 * wpt
    groups = wpt // L

    @pl.when(sid == 0)
    def _():
        pltpu.sync_copy(preds_hbm, preds_sh)

    plsc.subcore_barrier()
    tail = n_vars - vpt * (NS - 1)
    assert 0 < tail <= vpt and tail % 8 == 0

    @pl.when(sid < NS - 1)
    def _():
        pltpu.sync_copy(preds_sh.at[pl.ds(vpt * sid, vpt)], preds_local)

    @pl.when(sid == NS - 1)
    def _():
        pltpu.sync_copy(preds_sh.at[pl.ds(vpt * (NS - 1), tail)],
                        preds_local.at[pl.ds(0, tail)])

    powlo = jnp.left_shift(jnp.full((L,), 1, jnp.int32), iota)
    powhi = jnp.left_shift(jnp.full((L,), 1, jnp.int32), iota + L)

    def grp(g, carry):
        acc = jnp.zeros((L,), jnp.int32)
        for j in range(L):
            off = 2 * L * L * g + 2 * L * j
            b0 = preds_local[pl.ds(off, L)] >= 0.5
            b1 = preds_local[pl.ds(off + L, L)] >= 0.5
            w = (jnp.sum(jnp.where(b0, powlo, 0))
                 + jnp.sum(jnp.where(b1, powhi, 0)))
            acc = jnp.where(iota == j, w, acc)
        packed_local[pl.ds(L * g, L)] = acc
        return carry

    lax.fori_loop(0, groups, grp, 0)
    pltpu.sync_copy(packed_local, packed_sh.at[pl.ds(wpt * sid, wpt)])
    plsc.subcore_barrier()
    pltpu.sync_copy(packed_sh, packed_v)

    shift_idx = jnp.maximum(iota - 1, 0)
    last_idx = jnp.full((L,), L - 1, jnp.int32)

    def stream_step(slot, off, i, carry_vec, cnt_vec, first_vec):
        lit = lits_b[slot][pl.ds(off + i * L, L)]
        cls = cls_b[slot][pl.ds(off + i * L, L)]
        is_pos = lit < n_vars
        var = jnp.where(is_pos, lit, lit - n_vars)
        wd = plsc.load_gather(packed_v, [jnp.right_shift(var, 5)])
        bit = jnp.right_shift(wd, var & 31) & 1
        sat = (bit != 0) == is_pos
        m = jnp.where(sat, cls, -1)
        incl = plsc.cummax(m)
        shifted = jnp.take_along_axis(incl, shift_idx, axis=0,
                                      mode="promise_in_bounds")
        shifted = jnp.where(iota == 0, -1, shifted)
        excl = jnp.maximum(shifted, carry_vec)
        newc = sat & (cls > excl)
        cnt_vec = cnt_vec + newc.astype(jnp.int32)
        first_vec = jnp.minimum(first_vec, jnp.where(sat, cls, BIG))
        vmax = jnp.take_along_axis(incl, last_idx, axis=0,
                                   mode="promise_in_bounds")
        carry_vec = jnp.maximum(carry_vec, vmax)
        return carry_vec, cnt_vec, first_vec

    def compute(slot, state):
        def vec_body(i, st):
            ca, cb, cnt_vec, fa, fb = st
            ca, cnt_vec, fa = stream_step(slot, 0, i, ca, cnt_vec, fa)
            cb, cnt_vec, fb = stream_step(slot, CHUNK, i, cb, cnt_vec, fb)
            return ca, cb, cnt_vec, fa, fb

        return lax.fori_loop(0, CHUNK // L, vec_body, state, unroll=4)

    def one(c, slot, state):
        wait_chunk(slot)
        state = compute(slot, state)

        @pl.when(c + 2 < nchunk)
        def _():
            start_chunk(slot, c + 2)

        return state

    def pair_body(i, state):
        state = one(2 * i, 0, state)
        state = one(2 * i + 1, 1, state)
        return state

    init = (jnp.full((L,), -1, jnp.int32),
            jnp.full((L,), -1, jnp.int32),
            jnp.zeros((L,), jnp.int32),
            jnp.full((L,), BIG, jnp.int32),
            jnp.full((L,), BIG, jnp.int32))
    state = lax.fori_loop(0, nchunk // 2, pair_body, init)
    if nchunk % 2:
        state = one(nchunk - 1, 0, state)
    ca, cb, cnt_vec, fa, fb = state

    cnt = jnp.sum(cnt_vec)
    first_a = jnp.min(fa)
    last_a = jnp.max(ca)
    first_b = jnp.min(fb)
    last_b = jnp.max(cb)
    out = jnp.where(iota == 0, cnt,
                    jnp.where(iota == 1, first_a,
                              jnp.where(iota == 2, last_a,
                                        jnp.where(iota == 3, first_b,
                                                  jnp.where(iota == 4,
                                                            last_b, 0)))))
    outbuf_v[...] = out
    pltpu.sync_copy(outbuf_v, out_hbm.at[wid])


def _combine_body(n_vars, partials_ref, ncl_ref, o_ref):
    def body(t, st):
        total, m = st
        total = total + partials_ref[t, 0]
        fa = partials_ref[t, 1]
        la = partials_ref[t, 2]
        fb = partials_ref[t, 3]
        lb = partials_ref[t, 4]
        total = total - jnp.where(fa == m, jnp.int32(1), jnp.int32(0))
        m = jnp.maximum(m, la)
        total = total - jnp.where(fb == m, jnp.int32(1), jnp.int32(0))
        m = jnp.maximum(m, lb)
        return total, m

    total, _ = lax.fori_loop(0, NW, body, (jnp.int32(0), jnp.int32(-1)))
    o_ref[0, 0] = (ncl_ref[0, 0] - total.astype(jnp.float32)) / jnp.float32(n_vars)


def kernel(preds, lits, clauses, n_vars, n_clauses):
    del n_vars
    nv = preds.shape[0]
    nnz = lits.shape[0]
    per_tile = nnz // NW
    assert nnz % NW == 0 and per_tile % 2 == 0
    assert (per_tile // 2) % CHUNK == 0 and CHUNK % L == 0
    nwords = -(-nv // 32)
    wpt = -(-nwords // NS)
    wpt = -(-wpt // L) * L

    mesh = plsc.VectorSubcoreMesh(core_axis_name="c", subcore_axis_name="s")
    sc = functools.partial(
        pl.kernel,
        mesh=mesh,
        compiler_params=pltpu.CompilerParams(needs_layout_passes=False),
        out_type=jax.ShapeDtypeStruct((NW, L), jnp.int32),
        scratch_types=[
            pltpu.VMEM((32 * wpt,), jnp.float32),
            pltpu.VMEM((wpt,), jnp.int32),
            pltpu.VMEM((NS * wpt,), jnp.int32),
            pltpu.VMEM_SHARED((nv,), jnp.float32),
            pltpu.VMEM_SHARED((NS * wpt,), jnp.int32),
            pltpu.VMEM((2 * CHUNK,), jnp.int32),
            pltpu.VMEM((2 * CHUNK,), jnp.int32),
            pltpu.VMEM((2 * CHUNK,), jnp.int32),
            pltpu.VMEM((2 * CHUNK,), jnp.int32),
            pltpu.VMEM((L,), jnp.int32),
            pltpu.SemaphoreType.DMA,
            pltpu.SemaphoreType.DMA,
            pltpu.SemaphoreType.DMA,
            pltpu.SemaphoreType.DMA,
        ],
    )(functools.partial(_tile_body, n_vars=nv, per_tile=per_tile, wpt=wpt))
    partials = sc(preds, lits, clauses)

    ncl = jnp.asarray(n_clauses, jnp.float32).reshape(1, 1)
    out = pl.pallas_call(
        functools.partial(_combine_body, nv),
        in_specs=[pl.BlockSpec(memory_space=pltpu.SMEM),
                  pl.BlockSpec(memory_space=pltpu.SMEM)],
        out_specs=pl.BlockSpec(memory_space=pltpu.SMEM),
        out_shape=jax.ShapeDtypeStruct((1, 1), jnp.float32),
    )(partials, ncl)
    return out[0, 0]


# speedup vs baseline: 1.0448x; 1.0286x over previous
import functools

import jax
import jax.numpy as jnp
from jax import lax
from jax.experimental import pallas as pl
from jax.experimental.pallas import tpu as pltpu
from jax.experimental.pallas import tpu_sc as plsc

L = 16
NC = 2
NS = 16
NW = NC * NS
BIG = 0x3FFFFFFF
CHUNK = 2000


def _tile_body(preds_hbm, lits_hbm, clauses_hbm, out_hbm,
               preds_v, preds_sh, lits_b0, lits_b1, cls_b0, cls_b1, outbuf_v,
               sem_l0, sem_l1, sem_c0, sem_c1,
               *, n_vars, per_tile):
    sid = lax.axis_index("s")
    wid = sid * NC + lax.axis_index("c")
    half = per_tile // 2
    base_a = wid * per_tile
    base_b = base_a + half
    nchunk = half // CHUNK
    lits_b = (lits_b0, lits_b1)
    cls_b = (cls_b0, cls_b1)
    sem_l = (sem_l0, sem_l1)
    sem_c = (sem_c0, sem_c1)

    def start_chunk(slot, c):
        for src, bufs, sems in ((lits_hbm, lits_b, sem_l),
                                (clauses_hbm, cls_b, sem_c)):
            pltpu.make_async_copy(src.at[pl.ds(base_a + c * CHUNK, CHUNK)],
                                  bufs[slot].at[pl.ds(0, CHUNK)],
                                  sems[slot]).start()
            pltpu.make_async_copy(src.at[pl.ds(base_b + c * CHUNK, CHUNK)],
                                  bufs[slot].at[pl.ds(CHUNK, CHUNK)],
                                  sems[slot]).start()

    def wait_chunk(slot):
        for src, bufs, sems in ((lits_hbm, lits_b, sem_l),
                                (clauses_hbm, cls_b, sem_c)):
            pltpu.make_async_copy(src.at[pl.ds(0, CHUNK)],
                                  bufs[slot].at[pl.ds(0, CHUNK)],
                                  sems[slot]).wait()
            pltpu.make_async_copy(src.at[pl.ds(0, CHUNK)],
                                  bufs[slot].at[pl.ds(CHUNK, CHUNK)],
                                  sems[slot]).wait()

    start_chunk(0, 0)
    start_chunk(1, 1)

    @pl.when(sid == 0)
    def _():
        pltpu.sync_copy(preds_hbm, preds_sh)

    plsc.subcore_barrier()
    pltpu.sync_copy(preds_sh, preds_v)

    iota = lax.iota(jnp.int32, L)
    shift_idx = jnp.maximum(iota - 1, 0)
    last_idx = jnp.full((L,), L - 1, jnp.int32)

    def stream_step(slot, off, i, carry_vec, cnt_vec, first_vec):
        lit = lits_b[slot][pl.ds(off + i * L, L)]
        cls = cls_b[slot][pl.ds(off + i * L, L)]
        is_pos = lit < n_vars
        var = jnp.where(is_pos, lit, lit - n_vars)
        p = plsc.load_gather(preds_v, [var])
        sat = (p >= 0.5) == is_pos
        m = jnp.where(sat, cls, -1)
        incl = plsc.cummax(m)
        shifted = jnp.take_along_axis(incl, shift_idx, axis=0,
                                      mode="promise_in_bounds")
        shifted = jnp.where(iota == 0, -1, shifted)
        excl = jnp.maximum(shifted, carry_vec)
        newc = sat & (cls > excl)
        cnt_vec = cnt_vec + newc.astype(jnp.int32)
        first_vec = jnp.minimum(first_vec, jnp.where(sat, cls, BIG))
        vmax = jnp.take_along_axis(incl, last_idx, axis=0,
                                   mode="promise_in_bounds")
        carry_vec = jnp.maximum(carry_vec, vmax)
        return carry_vec, cnt_vec, first_vec

    def compute(slot, state):
        def vec_body(i, st):
            ca, cb, cnt_vec, fa, fb = st
            ca, cnt_vec, fa = stream_step(slot, 0, i, ca, cnt_vec, fa)
            cb, cnt_vec, fb = stream_step(slot, CHUNK, i, cb, cnt_vec, fb)
            return ca, cb, cnt_vec, fa, fb

        return lax.fori_loop(0, CHUNK // L, vec_body, state, unroll=4)

    def one(c, slot, state):
        wait_chunk(slot)
        state = compute(slot, state)

        @pl.when(c + 2 < nchunk)
        def _():
            start_chunk(slot, c + 2)

        return state

    def pair_body(i, state):
        state = one(2 * i, 0, state)
        state = one(2 * i + 1, 1, state)
        return state

    init = (jnp.full((L,), -1, jnp.int32),
            jnp.full((L,), -1, jnp.int32),
            jnp.zeros((L,), jnp.int32),
            jnp.full((L,), BIG, jnp.int32),
            jnp.full((L,), BIG, jnp.int32))
    state = lax.fori_loop(0, nchunk // 2, pair_body, init)
    if nchunk % 2:
        state = one(nchunk - 1, 0, state)
    ca, cb, cnt_vec, fa, fb = state

    cnt = jnp.sum(cnt_vec)
    first_a = jnp.min(fa)
    last_a = jnp.max(ca)
    first_b = jnp.min(fb)
    last_b = jnp.max(cb)
    out = jnp.where(iota == 0, cnt,
                    jnp.where(iota == 1, first_a,
                              jnp.where(iota == 2, last_a,
                                        jnp.where(iota == 3, first_b,
                                                  jnp.where(iota == 4,
                                                            last_b, 0)))))
    outbuf_v[...] = out
    pltpu.sync_copy(outbuf_v, out_hbm.at[wid])


def _combine_body(n_vars, partials_ref, ncl_ref, o_ref):
    def body(t, st):
        total, m = st
        total = total + partials_ref[t, 0]
        fa = partials_ref[t, 1]
        la = partials_ref[t, 2]
        fb = partials_ref[t, 3]
        lb = partials_ref[t, 4]
        total = total - jnp.where(fa == m, jnp.int32(1), jnp.int32(0))
        m = jnp.maximum(m, la)
        total = total - jnp.where(fb == m, jnp.int32(1), jnp.int32(0))
        m = jnp.maximum(m, lb)
        return total, m

    total, _ = lax.fori_loop(0, NW, body, (jnp.int32(0), jnp.int32(-1)))
    o_ref[0, 0] = (ncl_ref[0, 0] - total.astype(jnp.float32)) / jnp.float32(n_vars)


def kernel(preds, lits, clauses, n_vars, n_clauses):
    del n_vars
    nv = preds.shape[0]
    nnz = lits.shape[0]
    per_tile = nnz // NW
    assert nnz % NW == 0 and per_tile % 2 == 0
    assert (per_tile // 2) % CHUNK == 0 and CHUNK % L == 0

    mesh = plsc.VectorSubcoreMesh(core_axis_name="c", subcore_axis_name="s")
    sc = functools.partial(
        pl.kernel,
        mesh=mesh,
        compiler_params=pltpu.CompilerParams(needs_layout_passes=False),
        out_type=jax.ShapeDtypeStruct((NW, L), jnp.int32),
        scratch_types=[
            pltpu.VMEM((nv,), jnp.float32),
            pltpu.VMEM_SHARED((nv,), jnp.float32),
            pltpu.VMEM((2 * CHUNK,), jnp.int32),
            pltpu.VMEM((2 * CHUNK,), jnp.int32),
            pltpu.VMEM((2 * CHUNK,), jnp.int32),
            pltpu.VMEM((2 * CHUNK,), jnp.int32),
            pltpu.VMEM((L,), jnp.int32),
            pltpu.SemaphoreType.DMA,
            pltpu.SemaphoreType.DMA,
            pltpu.SemaphoreType.DMA,
            pltpu.SemaphoreType.DMA,
        ],
    )(functools.partial(_tile_body, n_vars=nv, per_tile=per_tile))
    partials = sc(preds, lits, clauses)

    ncl = jnp.asarray(n_clauses, jnp.float32).reshape(1, 1)
    out = pl.pallas_call(
        functools.partial(_combine_body, nv),
        in_specs=[pl.BlockSpec(memory_space=pltpu.SMEM),
                  pl.BlockSpec(memory_space=pltpu.SMEM)],
        out_specs=pl.BlockSpec(memory_space=pltpu.SMEM),
        out_shape=jax.ShapeDtypeStruct((1, 1), jnp.float32),
    )(partials, ncl)
    return out[0, 0]


# speedup vs baseline: 1.0475x; 1.0025x over previous
import functools

import jax
import jax.numpy as jnp
from jax import lax
from jax.experimental import pallas as pl
from jax.experimental.pallas import tpu as pltpu
from jax.experimental.pallas import tpu_sc as plsc

L = 16
NC = 2
NS = 16
NW = NC * NS
BIG = 0x3FFFFFFF
CHUNK = 2000
NBUF = 3


def _tile_body(preds_hbm, lits_hbm, clauses_hbm, out_hbm,
               preds_v, preds_sh,
               lits_b0, lits_b1, lits_b2, cls_b0, cls_b1, cls_b2, outbuf_v,
               sem0, sem1, sem2,
               *, n_vars, per_tile):
    sid = lax.axis_index("s")
    wid = sid * NC + lax.axis_index("c")
    base = wid * per_tile
    nchunk = per_tile // CHUNK
    lits_b = (lits_b0, lits_b1, lits_b2)
    cls_b = (cls_b0, cls_b1, cls_b2)
    sems = (sem0, sem1, sem2)

    def start_chunk(slot, c):
        off = base + c * CHUNK
        pltpu.make_async_copy(lits_hbm.at[pl.ds(off, CHUNK)],
                              lits_b[slot], sems[slot]).start()
        pltpu.make_async_copy(clauses_hbm.at[pl.ds(off, CHUNK)],
                              cls_b[slot], sems[slot]).start()

    def wait_chunk(slot):
        pltpu.make_async_copy(lits_hbm.at[pl.ds(0, CHUNK)],
                              lits_b[slot], sems[slot]).wait()
        pltpu.make_async_copy(clauses_hbm.at[pl.ds(0, CHUNK)],
                              cls_b[slot], sems[slot]).wait()

    for b in range(NBUF):
        start_chunk(b, b)

    @pl.when(sid == 0)
    def _():
        pltpu.sync_copy(preds_hbm, preds_sh)

    plsc.subcore_barrier()
    pltpu.sync_copy(preds_sh, preds_v)

    iota = lax.iota(jnp.int32, L)
    shift_idx = jnp.maximum(iota - 1, 0)
    last_idx = jnp.full((L,), L - 1, jnp.int32)

    def compute(slot, state):
        def vec_body(i, st):
            carry_vec, cnt_vec, first_vec = st
            lit = lits_b[slot][pl.ds(i * L, L)]
            cls = cls_b[slot][pl.ds(i * L, L)]
            is_pos = lit < n_vars
            var = jnp.where(is_pos, lit, lit - n_vars)
            p = plsc.load_gather(preds_v, [var])
            sat = (p >= 0.5) == is_pos
            m = jnp.where(sat, cls, -1)
            incl = plsc.cummax(m)
            shifted = jnp.take_along_axis(incl, shift_idx, axis=0,
                                          mode="promise_in_bounds")
            shifted = jnp.where(iota == 0, -1, shifted)
            excl = jnp.maximum(shifted, carry_vec)
            newc = sat & (cls > excl)
            cnt_vec = cnt_vec + newc.astype(jnp.int32)
            first_vec = jnp.minimum(first_vec, jnp.where(sat, cls, BIG))
            vmax = jnp.take_along_axis(incl, last_idx, axis=0,
                                       mode="promise_in_bounds")
            carry_vec = jnp.maximum(carry_vec, vmax)
            return carry_vec, cnt_vec, first_vec

        return lax.fori_loop(0, CHUNK // L, vec_body, state, unroll=4)

    def one(c, slot, state):
        wait_chunk(slot)
        state = compute(slot, state)

        @pl.when(c + NBUF < nchunk)
        def _():
            start_chunk(slot, c + NBUF)

        return state

    def ring_body(i, state):
        for b in range(NBUF):
            state = one(NBUF * i + b, b, state)
        return state

    init = (jnp.full((L,), -1, jnp.int32),
            jnp.zeros((L,), jnp.int32),
            jnp.full((L,), BIG, jnp.int32))
    state = lax.fori_loop(0, nchunk // NBUF, ring_body, init)
    for r in range(nchunk % NBUF):
        state = one(NBUF * (nchunk // NBUF) + r, r, state)
    carry_vec, cnt_vec, first_vec = state

    cnt = jnp.sum(cnt_vec)
    first = jnp.min(first_vec)
    last = jnp.max(carry_vec)
    out = jnp.where(iota == 0, cnt,
                    jnp.where(iota == 1, first,
                              jnp.where(iota == 2, last, 0)))
    outbuf_v[...] = out
    pltpu.sync_copy(outbuf_v, out_hbm.at[wid])


def _combine_body(n_vars, partials_ref, ncl_ref, o_ref):
    def body(t, st):
        total, m = st
        c = partials_ref[t, 0]
        f = partials_ref[t, 1]
        l = partials_ref[t, 2]
        dup = jnp.where(f == m, jnp.int32(1), jnp.int32(0))
        return total + c - dup, jnp.maximum(m, l)

    total, _ = lax.fori_loop(0, NW, body, (jnp.int32(0), jnp.int32(-1)))
    o_ref[0, 0] = (ncl_ref[0, 0] - total.astype(jnp.float32)) / jnp.float32(n_vars)


def kernel(preds, lits, clauses, n_vars, n_clauses):
    del n_vars
    nv = preds.shape[0]
    nnz = lits.shape[0]
    per_tile = nnz // NW
    assert nnz % NW == 0
    assert per_tile % CHUNK == 0 and CHUNK % L == 0

    mesh = plsc.VectorSubcoreMesh(core_axis_name="c", subcore_axis_name="s")
    sc = functools.partial(
        pl.kernel,
        mesh=mesh,
        compiler_params=pltpu.CompilerParams(needs_layout_passes=False),
        out_type=jax.ShapeDtypeStruct((NW, L), jnp.int32),
        scratch_types=[
            pltpu.VMEM((nv,), jnp.float32),
            pltpu.VMEM_SHARED((nv,), jnp.float32),
            pltpu.VMEM((CHUNK,), jnp.int32),
            pltpu.VMEM((CHUNK,), jnp.int32),
            pltpu.VMEM((CHUNK,), jnp.int32),
            pltpu.VMEM((CHUNK,), jnp.int32),
            pltpu.VMEM((CHUNK,), jnp.int32),
            pltpu.VMEM((CHUNK,), jnp.int32),
            pltpu.VMEM((L,), jnp.int32),
            pltpu.SemaphoreType.DMA,
            pltpu.SemaphoreType.DMA,
            pltpu.SemaphoreType.DMA,
        ],
    )(functools.partial(_tile_body, n_vars=nv, per_tile=per_tile))
    partials = sc(preds, lits, clauses)

    ncl = jnp.asarray(n_clauses, jnp.float32).reshape(1, 1)
    out = pl.pallas_call(
        functools.partial(_combine_body, nv),
        in_specs=[pl.BlockSpec(memory_space=pltpu.SMEM),
                  pl.BlockSpec(memory_space=pltpu.SMEM)],
        out_specs=pl.BlockSpec(memory_space=pltpu.SMEM),
        out_shape=jax.ShapeDtypeStruct((1, 1), jnp.float32),
    )(partials, ncl)
    return out[0, 0]
